# Initial kernel scaffold; baseline (speedup 1.0000x reference)
import jax
import jax.numpy as jnp
from jax.experimental import pallas as pl


def kernel(x, storage):
    raise NotImplementedError("write your pallas kernel here")


# speedup vs baseline: 6.4084x; 6.4084x over previous
import functools

import jax
import jax.numpy as jnp
from jax import lax
from jax.experimental import pallas as pl
from jax.experimental.pallas import tpu as pltpu
from jax.experimental.pallas import tpu_sc as plsc

_KD = 128
_VD = 128
_K = 32
_S = 100000
_B = 4096
_NCH = 784
_SP = _NCH * 128
_ST = 1024
_BT = 512
_NEG = -1e30


def _p0_body(x_ref, xn_ref):
    x = x_ref[...]
    n = jnp.sqrt(jnp.sum(x * x, axis=1, keepdims=True))
    xn_ref[...] = x / jnp.maximum(n, 1e-12)


def _p1_body(xn_ref, k_ref, sims_ref, cmax_ref):
    j = pl.program_id(0)
    k = k_ref[...]
    n = jnp.sqrt(jnp.sum(k * k, axis=1, keepdims=True))
    kn = k / jnp.maximum(n, 1e-12)
    s = lax.dot_general(
        xn_ref[...], kn, (((1,), (1,)), ((), ())),
        preferred_element_type=jnp.float32,
        precision=lax.Precision.DEFAULT,
    )
    sims_ref[...] = s
    cmax_ref[...] = jnp.max(
        s.reshape(s.shape[0], s.shape[1] // 128, 128), axis=2).T

    @pl.when(j == pl.num_programs(0) - 1)
    def _():
        col = lax.broadcasted_iota(jnp.int32, s.shape, 1) + j * s.shape[1]
        s2 = jnp.where(col < _S, s, _NEG)
        sims_ref[...] = s2
        cmax_ref[...] = jnp.max(
            s2.reshape(s2.shape[0], s2.shape[1] // 128, 128), axis=2).T


def _p2_body(cm_ref, cid_ref, fidx_ref):
    i = pl.program_id(0)
    vals = cm_ref[...]
    iota = lax.broadcasted_iota(jnp.int32, vals.shape, 0)
    ids = []
    for _ in range(_K):
        m = jnp.max(vals, axis=0, keepdims=True)
        cand = jnp.where(vals == m, iota, jnp.int32(1 << 30))
        idx = jnp.min(cand, axis=0, keepdims=True)
        ids.append(idx)
        vals = jnp.where(iota == idx, _NEG, vals)
    cid = jnp.concatenate(ids, axis=0)
    cid_ref[...] = cid
    row = i * vals.shape[1] + lax.broadcasted_iota(jnp.int32, cid.shape, 1)
    fidx_ref[...] = row * _NCH + cid


def _p4_body(c_ref, cid_ref, w_ref, vidx_ref):
    vals = c_ref[...]
    cid = cid_ref[...]
    k_iota = lax.broadcasted_iota(jnp.int32, vals.shape, 0)
    l_iota = lax.broadcasted_iota(jnp.int32, vals.shape, 2)
    pos = k_iota * 128 + l_iota
    slot_iota = lax.broadcasted_iota(jnp.int32, cid.shape, 0)
    ms, gcols = [], []
    for _ in range(_K):
        m = jnp.max(jnp.max(vals, axis=2), axis=0, keepdims=True)
        sel = jnp.where(vals == m[:, :, None], pos, jnp.int32(1 << 30))
        p = jnp.min(jnp.min(sel, axis=2), axis=0, keepdims=True)
        ms.append(m)
        cs = lax.shift_right_logical(p, 7)
        lane = jnp.bitwise_and(p, 127)
        g = jnp.sum(jnp.where(slot_iota == cs, cid, 0), axis=0, keepdims=True)
        gcols.append(g * 128 + lane)
        vals = jnp.where(pos == p[:, :, None], _NEG, vals)
    msv = jnp.concatenate(ms, axis=0)
    ex = jnp.exp(msv - msv[0:1])
    w_ref[...] = ex / jnp.sum(ex, axis=0, keepdims=True)
    vidx_ref[...] = jnp.concatenate(gcols, axis=0)


def _p6_body(w_ref, v_ref, o_ref):
    w = w_ref[...]
    v = v_ref[...]
    o_ref[...] = jnp.sum(w[:, :, None] * v, axis=0)


def _gather_rows(table, idx_flat, width):
    n_rows = idx_flat.shape[0]
    nc, ns = 2, 16
    nw = nc * ns
    per_w = n_rows // nw
    ch = 128
    n_it = per_w // ch
    mesh = plsc.VectorSubcoreMesh(core_axis_name="c", subcore_axis_name="s")

    @functools.partial(
        pl.kernel, mesh=mesh,
        out_type=jax.ShapeDtypeStruct((n_rows, width), jnp.float32),
        scratch_types=[
            pltpu.VMEM((ch,), jnp.int32),
            pltpu.VMEM((ch, width), jnp.float32),
            pltpu.SemaphoreType.DMA,
        ],
    )
    def k(table_hbm, idx_hbm, out_hbm, idx_v, rows_v, sem):
        wid = lax.axis_index("s") * nc + lax.axis_index("c")
        base = wid * per_w

        def body(i, carry):
            off = base + i * ch
            pltpu.sync_copy(idx_hbm.at[pl.ds(off, ch)], idx_v)
            pltpu.async_copy(table_hbm.at[idx_v], rows_v, sem).wait()
            pltpu.sync_copy(rows_v, out_hbm.at[pl.ds(off, ch)])
            return carry

        lax.fori_loop(0, n_it, body, 0)

    return k(table, idx_flat)


def kernel(x, storage):
    nbt = _B // _BT
    storage_p = jnp.zeros((_SP, _KD + _VD), jnp.float32).at[:_S, :].set(storage)

    xn = pl.pallas_call(
        _p0_body,
        in_specs=[pl.BlockSpec((_B, _KD), lambda: (0, 0))],
        out_specs=pl.BlockSpec((_B, _KD), lambda: (0, 0)),
        out_shape=jax.ShapeDtypeStruct((_B, _KD), jnp.float32),
    )(x)

    sims, cmax = pl.pallas_call(
        _p1_body,
        grid=(_SP // _ST, nbt),
        in_specs=[
            pl.BlockSpec((_BT, _KD), lambda j, i: (i, 0)),
            pl.BlockSpec((_ST, _KD), lambda j, i: (j, 0)),
        ],
        out_specs=[
            pl.BlockSpec((_BT, _ST), lambda j, i: (i, j)),
            pl.BlockSpec((_ST // 128, _BT), lambda j, i: (j, i)),
        ],
        out_shape=[
            jax.ShapeDtypeStruct((_B, _SP), jnp.float32),
            jax.ShapeDtypeStruct((_NCH, _B), jnp.float32),
        ],
    )(xn, storage_p)

    cid, fidx = pl.pallas_call(
        _p2_body,
        grid=(nbt,),
        in_specs=[pl.BlockSpec((_NCH, _BT), lambda i: (0, i))],
        out_specs=[
            pl.BlockSpec((_K, _BT), lambda i: (0, i)),
            pl.BlockSpec((_K, _BT), lambda i: (0, i)),
        ],
        out_shape=[
            jax.ShapeDtypeStruct((_K, _B), jnp.int32),
            jax.ShapeDtypeStruct((_K, _B), jnp.int32),
        ],
    )(cmax)

    cand = _gather_rows(
        sims.reshape(_B * _NCH, 128), fidx.reshape(_K * _B), 128)

    w, vidx = pl.pallas_call(
        _p4_body,
        grid=(nbt,),
        in_specs=[
            pl.BlockSpec((_K, _BT, 128), lambda i: (0, i, 0)),
            pl.BlockSpec((_K, _BT), lambda i: (0, i)),
        ],
        out_specs=[
            pl.BlockSpec((_K, _BT), lambda i: (0, i)),
            pl.BlockSpec((_K, _BT), lambda i: (0, i)),
        ],
        out_shape=[
            jax.ShapeDtypeStruct((_K, _B), jnp.float32),
            jax.ShapeDtypeStruct((_K, _B), jnp.int32),
        ],
    )(cand.reshape(_K, _B, 128), cid)

    vals = _gather_rows(storage_p, vidx.reshape(_K * _B), _KD + _VD)

    out = pl.pallas_call(
        _p6_body,
        grid=(nbt,),
        in_specs=[
            pl.BlockSpec((_K, _BT), lambda i: (0, i)),
            pl.BlockSpec((_K, _BT, _VD), lambda i: (0, i, 1)),
        ],
        out_specs=pl.BlockSpec((_BT, _VD), lambda i: (i, 0)),
        out_shape=jax.ShapeDtypeStruct((_B, _VD), jnp.float32),
    )(w, vals.reshape(_K, _B, _KD + _VD))

    return out


# speedup vs baseline: 6.7897x; 1.0595x over previous
import functools

import jax
import jax.numpy as jnp
from jax import lax
from jax.experimental import pallas as pl
from jax.experimental.pallas import tpu as pltpu
from jax.experimental.pallas import tpu_sc as plsc

_KD = 128
_VD = 128
_K = 32
_S = 100000
_B = 4096
_NCH = 784
_SP = _NCH * 128
_ST = 2048
_BT = 512
_NEG = -1e30


def _p0_body(x_ref, xn_ref):
    x = x_ref[...]
    n = jnp.sqrt(jnp.sum(x * x, axis=1, keepdims=True))
    xn_ref[...] = x / jnp.maximum(n, 1e-12)


def _p1_body(xn_ref, k_ref, sims_ref, cmax_ref, kn_ref):
    j = pl.program_id(0)
    i = pl.program_id(1)

    @pl.when(i == 0)
    def _():
        k = k_ref[...]
        n = jnp.sqrt(jnp.sum(k * k, axis=1, keepdims=True))
        kn_ref[...] = k / jnp.maximum(n, 1e-12)

    s = lax.dot_general(
        xn_ref[...], kn_ref[...], (((1,), (1,)), ((), ())),
        preferred_element_type=jnp.float32,
        precision=lax.Precision.DEFAULT,
    )
    sims_ref[...] = s
    cmax_ref[...] = jnp.max(
        s.reshape(s.shape[0], s.shape[1] // 128, 128), axis=2).T

    @pl.when(j == pl.num_programs(0) - 1)
    def _():
        col = lax.broadcasted_iota(jnp.int32, s.shape, 1) + j * s.shape[1]
        s2 = jnp.where(col < _S, s, _NEG)
        sims_ref[...] = s2
        cmax_ref[...] = jnp.max(
            s2.reshape(s2.shape[0], s2.shape[1] // 128, 128), axis=2).T


def _p2_body(cm_ref, cid_ref, fidx_ref):
    i = pl.program_id(0)
    vals = cm_ref[...]
    iota = lax.broadcasted_iota(jnp.int32, vals.shape, 0)
    ids = []
    for _ in range(_K):
        m = jnp.max(vals, axis=0, keepdims=True)
        cand = jnp.where(vals == m, iota, jnp.int32(1 << 30))
        idx = jnp.min(cand, axis=0, keepdims=True)
        ids.append(idx)
        vals = jnp.where(iota == idx, _NEG, vals)
    cid = jnp.concatenate(ids, axis=0)
    cid_ref[...] = cid
    row = i * vals.shape[1] + lax.broadcasted_iota(jnp.int32, cid.shape, 1)
    fidx_ref[...] = row * _NCH + cid


def _p4_body(c_ref, cid_ref, w_ref, vidx_ref):
    vals = c_ref[...]
    cid = cid_ref[...]
    k_iota = lax.broadcasted_iota(jnp.int32, vals.shape, 0)
    l_iota = lax.broadcasted_iota(jnp.int32, vals.shape, 2)
    pos = k_iota * 128 + l_iota
    slot_iota = lax.broadcasted_iota(jnp.int32, cid.shape, 0)
    ms, gcols = [], []
    for _ in range(_K):
        m = jnp.max(jnp.max(vals, axis=2), axis=0, keepdims=True)
        sel = jnp.where(vals == m[:, :, None], pos, jnp.int32(1 << 30))
        p = jnp.min(jnp.min(sel, axis=2), axis=0, keepdims=True)
        ms.append(m)
        cs = lax.shift_right_logical(p, 7)
        lane = jnp.bitwise_and(p, 127)
        g = jnp.sum(jnp.where(slot_iota == cs, cid, 0), axis=0, keepdims=True)
        gcols.append(g * 128 + lane)
        vals = jnp.where(pos == p[:, :, None], _NEG, vals)
    msv = jnp.concatenate(ms, axis=0)
    ex = jnp.exp(msv - msv[0:1])
    w_ref[...] = ex / jnp.sum(ex, axis=0, keepdims=True)
    vidx_ref[...] = jnp.concatenate(gcols, axis=0)


def _p6_body(w_ref, v_ref, o_ref):
    w = w_ref[...]
    v = v_ref[...]
    o_ref[...] = jnp.sum(w[:, :, None] * v, axis=0)


def _gather_rows(table, idx_flat, width):
    n_rows = idx_flat.shape[0]
    nc, ns = 2, 16
    nw = nc * ns
    per_w = n_rows // nw
    ch = 128
    n_it = per_w // ch
    mesh = plsc.VectorSubcoreMesh(core_axis_name="c", subcore_axis_name="s")

    @functools.partial(
        pl.kernel, mesh=mesh,
        out_type=jax.ShapeDtypeStruct((n_rows, width), jnp.float32),
        scratch_types=[
            pltpu.VMEM((ch,), jnp.int32),
            pltpu.VMEM((ch, width), jnp.float32),
            pltpu.SemaphoreType.DMA,
        ],
    )
    def k(table_hbm, idx_hbm, out_hbm, idx_v, rows_v, sem):
        wid = lax.axis_index("s") * nc + lax.axis_index("c")
        base = wid * per_w

        def body(i, carry):
            off = base + i * ch
            pltpu.sync_copy(idx_hbm.at[pl.ds(off, ch)], idx_v)
            pltpu.async_copy(table_hbm.at[idx_v], rows_v, sem).wait()
            pltpu.sync_copy(rows_v, out_hbm.at[pl.ds(off, ch)])
            return carry

        lax.fori_loop(0, n_it, body, 0)

    return k(table, idx_flat)


def kernel(x, storage):
    nbt = _B // _BT
    storage_p = jnp.zeros((_SP, _KD + _VD), jnp.float32).at[:_S, :].set(storage)

    xn = pl.pallas_call(
        _p0_body,
        in_specs=[pl.BlockSpec((_B, _KD), lambda: (0, 0))],
        out_specs=pl.BlockSpec((_B, _KD), lambda: (0, 0)),
        out_shape=jax.ShapeDtypeStruct((_B, _KD), jnp.float32),
    )(x)

    sims, cmax = pl.pallas_call(
        _p1_body,
        grid=(_SP // _ST, nbt),
        in_specs=[
            pl.BlockSpec((_BT, _KD), lambda j, i: (i, 0)),
            pl.BlockSpec((_ST, _KD), lambda j, i: (j, 0)),
        ],
        out_specs=[
            pl.BlockSpec((_BT, _ST), lambda j, i: (i, j)),
            pl.BlockSpec((_ST // 128, _BT), lambda j, i: (j, i)),
        ],
        out_shape=[
            jax.ShapeDtypeStruct((_B, _SP), jnp.float32),
            jax.ShapeDtypeStruct((_NCH, _B), jnp.float32),
        ],
        scratch_shapes=[pltpu.VMEM((_ST, _KD), jnp.float32)],
    )(xn, storage_p)

    cid, fidx = pl.pallas_call(
        _p2_body,
        grid=(nbt,),
        in_specs=[pl.BlockSpec((_NCH, _BT), lambda i: (0, i))],
        out_specs=[
            pl.BlockSpec((_K, _BT), lambda i: (0, i)),
            pl.BlockSpec((_K, _BT), lambda i: (0, i)),
        ],
        out_shape=[
            jax.ShapeDtypeStruct((_K, _B), jnp.int32),
            jax.ShapeDtypeStruct((_K, _B), jnp.int32),
        ],
    )(cmax)

    cand = _gather_rows(
        sims.reshape(_B * _NCH, 128), fidx.reshape(_K * _B), 128)

    w, vidx = pl.pallas_call(
        _p4_body,
        grid=(nbt,),
        in_specs=[
            pl.BlockSpec((_K, _BT, 128), lambda i: (0, i, 0)),
            pl.BlockSpec((_K, _BT), lambda i: (0, i)),
        ],
        out_specs=[
            pl.BlockSpec((_K, _BT), lambda i: (0, i)),
            pl.BlockSpec((_K, _BT), lambda i: (0, i)),
        ],
        out_shape=[
            jax.ShapeDtypeStruct((_K, _B), jnp.float32),
            jax.ShapeDtypeStruct((_K, _B), jnp.int32),
        ],
    )(cand.reshape(_K, _B, 128), cid)

    vals = _gather_rows(storage_p, vidx.reshape(_K * _B), _KD + _VD)

    out = pl.pallas_call(
        _p6_body,
        grid=(nbt,),
        in_specs=[
            pl.BlockSpec((_K, _BT), lambda i: (0, i)),
            pl.BlockSpec((_K, _BT, _VD), lambda i: (0, i, 1)),
        ],
        out_specs=pl.BlockSpec((_BT, _VD), lambda i: (i, 0)),
        out_shape=jax.ShapeDtypeStruct((_B, _VD), jnp.float32),
    )(w, vals.reshape(_K, _B, _KD + _VD))

    return out
